# baseline (speedup 1.0000x reference)
import functools

import jax
import jax.numpy as jnp
from jax import lax
from jax.experimental import pallas as pl
from jax.experimental.pallas import tpu as pltpu
from jax.experimental.pallas import tpu_sc as plsc

_N, _E, _D = 10000, 320000, 128
_NC, _NS = 2, 16
_NW = _NC * _NS
_K = 128
_NCHUNK = 2560
_EPAD = _NCHUNK * _K
_RPAD = 10112
_RPT = _RPAD // _NS
_SLOW_CID = 0
_F_SLOW = 32
_F_FAST = 128
_F_MAX = _F_FAST


def _mlp1(x, W1, b1):
    blk = 1000

    def body(x_ref, w_ref, b_ref, o_ref):
        h = jnp.dot(x_ref[...], w_ref[...], preferred_element_type=jnp.float32)
        o_ref[...] = jnp.maximum(h + b_ref[...], 0.0)

    return pl.pallas_call(
        body,
        grid=(_N // blk,),
        in_specs=[
            pl.BlockSpec((blk, _D), lambda i: (i, 0)),
            pl.BlockSpec((_D, _D), lambda i: (0, 0)),
            pl.BlockSpec((1, _D), lambda i: (0, 0)),
        ],
        out_specs=pl.BlockSpec((blk, _D), lambda i: (i, 0)),
        out_shape=jax.ShapeDtypeStruct((_N, _D), jnp.float32),
    )(x, W1, b1.reshape(1, _D))


def _sc_aggregate(h, row_p, col_p, zeros):
    mesh = plsc.VectorSubcoreMesh(core_axis_name="c", subcore_axis_name="s")

    @functools.partial(
        pl.kernel,
        mesh=mesh,
        out_type=jax.ShapeDtypeStruct((_NC, _RPAD, _D), jnp.float32),
        scratch_types=[
            pltpu.VMEM((_F_MAX, _K), jnp.int32),
            pltpu.VMEM((_K,), jnp.int32),
            pltpu.VMEM((_K,), jnp.int32),
            pltpu.VMEM((_K, _D), jnp.float32),
            pltpu.VMEM((_K, _D), jnp.float32),
            pltpu.VMEM_SHARED((_RPAD, _D), jnp.float32),
            pltpu.SemaphoreType.DMA,
            pltpu.SemaphoreType.DMA,
            pltpu.SemaphoreType.DMA,
            pltpu.SemaphoreType.DMA,
            pltpu.SemaphoreType.DMA,
            pltpu.SemaphoreType.DMA,
        ],
    )
    def agg_kernel(h_hbm, row_hbm, col_hbm, z_hbm, out_hbm,
                   colv, rowb0, rowb1, rows0, rows1, acc_sh,
                   gsem0, gsem1, ssem0, ssem1, rsem0, rsem1):
        cid = lax.axis_index("c")
        sid = lax.axis_index("s")
        on_slow = cid == _SLOW_CID
        fh = jnp.where(on_slow, _F_SLOW // 2, _F_FAST // 2)
        base = jnp.where(on_slow, sid * _F_SLOW,
                         _NS * _F_SLOW + sid * _F_FAST)
        pltpu.sync_copy(z_hbm.at[pl.ds(sid * _RPT, _RPT)],
                        acc_sh.at[pl.ds(sid * _RPT, _RPT)])
        pltpu.sync_copy(col_hbm.at[pl.ds(base, _F_MAX)], colv)
        plsc.subcore_barrier()

        def rowload(c, rowb, sem):
            return pltpu.async_copy(row_hbm.at[base + c, 0], rowb, sem)

        def rowload_wait(c, rowb, sem):
            pltpu.make_async_copy(row_hbm.at[base + c, 0], rowb, sem).wait()

        def gather(c, rows, sem):
            return pltpu.async_copy(h_hbm.at[colv.at[c]], rows, sem)

        def gather_wait(c, rows, sem):
            pltpu.make_async_copy(h_hbm.at[colv.at[c]], rows, sem).wait()

        def scatter(rowb, rows, sem):
            return pltpu.async_copy(rows, acc_sh.at[rowb], sem, add=True)

        def scatter_wait(rowb, rows, sem):
            pltpu.make_async_copy(rows, acc_sh.at[rowb], sem).wait()

        rowload(0, rowb0, rsem0)
        rowload(1, rowb1, rsem1)
        gather(0, rows0, gsem0)

        def body(i, carry):
            c0 = 2 * i
            c1 = c0 + 1

            @pl.when(i < fh)
            def _():
                gather_wait(c0, rows0, gsem0)
                gather(c1, rows1, gsem1)
                rowload_wait(c0, rowb0, rsem0)
                scatter(rowb0, rows0, ssem0)
                gather_wait(c1, rows1, gsem1)
                scatter_wait(rowb0, rows0, ssem0)

                @pl.when(i < fh - 1)
                def _():
                    rowload(c0 + 2, rowb0, rsem0)
                    gather(c0 + 2, rows0, gsem0)

                rowload_wait(c1, rowb1, rsem1)
                scatter(rowb1, rows1, ssem1)
                scatter_wait(rowb1, rows1, ssem1)

                @pl.when(i < fh - 1)
                def _():
                    rowload(c1 + 2, rowb1, rsem1)

            return carry

        lax.fori_loop(0, _F_MAX // 2, body, 0)
        plsc.subcore_barrier()
        pltpu.sync_copy(acc_sh.at[pl.ds(sid * _RPT, _RPT)],
                        out_hbm.at[cid, pl.ds(sid * _RPT, _RPT)])

    return agg_kernel(h, row_p, col_p, zeros)


def _mlp2(a0, a1, W2, b2):
    blk = 1000

    def body(a0_ref, a1_ref, w_ref, b_ref, o_ref):
        agg = a0_ref[...] + a1_ref[...]
        out = jnp.dot(agg, w_ref[...], preferred_element_type=jnp.float32)
        out = out + b_ref[...]
        m = jnp.max(out, axis=1, keepdims=True)
        lse = jnp.log(jnp.sum(jnp.exp(out - m), axis=1, keepdims=True)) + m
        o_ref[...] = out - lse

    return pl.pallas_call(
        body,
        grid=(_N // blk,),
        in_specs=[
            pl.BlockSpec((blk, _D), lambda i: (i, 0)),
            pl.BlockSpec((blk, _D), lambda i: (i, 0)),
            pl.BlockSpec((_D, _D), lambda i: (0, 0)),
            pl.BlockSpec((1, _D), lambda i: (0, 0)),
        ],
        out_specs=pl.BlockSpec((blk, _D), lambda i: (i, 0)),
        out_shape=jax.ShapeDtypeStruct((_N, _D), jnp.float32),
    )(a0, a1, W2, b2.reshape(1, _D))


def kernel(x, adj_or_edge_index, W1, b1, W2, b2):
    row = adj_or_edge_index[0]
    col = adj_or_edge_index[1]
    pad = _EPAD - _E
    row_p = jnp.concatenate([row, jnp.full((pad,), _N, jnp.int32)])
    col_p = jnp.concatenate([col, jnp.zeros((pad,), jnp.int32)])
    row_p = row_p.reshape(_NCHUNK, 1, _K)
    col_p = col_p.reshape(_NCHUNK, _K)
    h = _mlp1(x, W1, b1)
    zeros = jnp.zeros((_RPAD, _D), jnp.float32)
    agg = _sc_aggregate(h, row_p, col_p, zeros)
    return _mlp2(agg[0, :_N], agg[1, :_N], W2, b2)


# speedup vs baseline: 1.0293x; 1.0293x over previous
import functools

import jax
import jax.numpy as jnp
from jax import lax
from jax.experimental import pallas as pl
from jax.experimental.pallas import tpu as pltpu
from jax.experimental.pallas import tpu_sc as plsc

_N, _E, _D = 10000, 320000, 128
_NC, _NS = 2, 16
_NW = _NC * _NS
_K = 128
_NCHUNK = 2560
_EPAD = _NCHUNK * _K
_RPAD = 10112
_RPT = _RPAD // _NS
_SLOW_CID = 1
_F_SLOW = 32
_F_FAST = 128
_F_MAX = _F_FAST


def _mlp1(x, W1, b1):
    blk = 1000

    def body(x_ref, w_ref, b_ref, o_ref):
        h = jnp.dot(x_ref[...], w_ref[...], preferred_element_type=jnp.float32)
        o_ref[...] = jnp.maximum(h + b_ref[...], 0.0)

    return pl.pallas_call(
        body,
        grid=(_N // blk,),
        in_specs=[
            pl.BlockSpec((blk, _D), lambda i: (i, 0)),
            pl.BlockSpec((_D, _D), lambda i: (0, 0)),
            pl.BlockSpec((1, _D), lambda i: (0, 0)),
        ],
        out_specs=pl.BlockSpec((blk, _D), lambda i: (i, 0)),
        out_shape=jax.ShapeDtypeStruct((_N, _D), jnp.float32),
    )(x, W1, b1.reshape(1, _D))


def _sc_aggregate(h, row_p, col_p, zeros):
    mesh = plsc.VectorSubcoreMesh(core_axis_name="c", subcore_axis_name="s")

    @functools.partial(
        pl.kernel,
        mesh=mesh,
        out_type=jax.ShapeDtypeStruct((_NC, _RPAD, _D), jnp.float32),
        scratch_types=[
            pltpu.VMEM((_F_MAX, _K), jnp.int32),
            pltpu.VMEM((_K,), jnp.int32),
            pltpu.VMEM((_K,), jnp.int32),
            pltpu.VMEM((_K, _D), jnp.float32),
            pltpu.VMEM((_K, _D), jnp.float32),
            pltpu.VMEM_SHARED((_RPAD, _D), jnp.float32),
            pltpu.SemaphoreType.DMA,
            pltpu.SemaphoreType.DMA,
            pltpu.SemaphoreType.DMA,
            pltpu.SemaphoreType.DMA,
            pltpu.SemaphoreType.DMA,
            pltpu.SemaphoreType.DMA,
        ],
    )
    def agg_kernel(h_hbm, row_hbm, col_hbm, z_hbm, out_hbm,
                   colv, rowb0, rowb1, rows0, rows1, acc_sh,
                   gsem0, gsem1, ssem0, ssem1, rsem0, rsem1):
        cid = lax.axis_index("c")
        sid = lax.axis_index("s")
        on_slow = cid == _SLOW_CID
        fh = jnp.where(on_slow, _F_SLOW // 2, _F_FAST // 2)
        base = jnp.where(on_slow, sid * _F_SLOW,
                         _NS * _F_SLOW + sid * _F_FAST)
        pltpu.sync_copy(z_hbm.at[pl.ds(sid * _RPT, _RPT)],
                        acc_sh.at[pl.ds(sid * _RPT, _RPT)])
        pltpu.sync_copy(col_hbm.at[pl.ds(base, _F_MAX)], colv)
        plsc.subcore_barrier()

        def rowload(c, rowb, sem):
            return pltpu.async_copy(row_hbm.at[base + c, 0], rowb, sem)

        def rowload_wait(c, rowb, sem):
            pltpu.make_async_copy(row_hbm.at[base + c, 0], rowb, sem).wait()

        def gather(c, rows, sem):
            return pltpu.async_copy(h_hbm.at[colv.at[c]], rows, sem)

        def gather_wait(c, rows, sem):
            pltpu.make_async_copy(h_hbm.at[colv.at[c]], rows, sem).wait()

        def scatter(rowb, rows, sem):
            return pltpu.async_copy(rows, acc_sh.at[rowb], sem, add=True)

        def scatter_wait(rowb, rows, sem):
            pltpu.make_async_copy(rows, acc_sh.at[rowb], sem).wait()

        rowload(0, rowb0, rsem0)
        rowload(1, rowb1, rsem1)
        gather(0, rows0, gsem0)

        def body(i, carry):
            c0 = 2 * i
            c1 = c0 + 1

            @pl.when(i < fh)
            def _():
                gather_wait(c0, rows0, gsem0)
                gather(c1, rows1, gsem1)
                rowload_wait(c0, rowb0, rsem0)
                scatter(rowb0, rows0, ssem0)
                gather_wait(c1, rows1, gsem1)
                scatter_wait(rowb0, rows0, ssem0)

                @pl.when(i < fh - 1)
                def _():
                    rowload(c0 + 2, rowb0, rsem0)
                    gather(c0 + 2, rows0, gsem0)

                rowload_wait(c1, rowb1, rsem1)
                scatter(rowb1, rows1, ssem1)
                scatter_wait(rowb1, rows1, ssem1)

                @pl.when(i < fh - 1)
                def _():
                    rowload(c1 + 2, rowb1, rsem1)

            return carry

        lax.fori_loop(0, _F_MAX // 2, body, 0)
        plsc.subcore_barrier()
        pltpu.sync_copy(acc_sh.at[pl.ds(sid * _RPT, _RPT)],
                        out_hbm.at[cid, pl.ds(sid * _RPT, _RPT)])

    return agg_kernel(h, row_p, col_p, zeros)


def _mlp2(a0, a1, W2, b2):
    blk = 1000

    def body(a0_ref, a1_ref, w_ref, b_ref, o_ref):
        agg = a0_ref[...] + a1_ref[...]
        out = jnp.dot(agg, w_ref[...], preferred_element_type=jnp.float32)
        out = out + b_ref[...]
        m = jnp.max(out, axis=1, keepdims=True)
        lse = jnp.log(jnp.sum(jnp.exp(out - m), axis=1, keepdims=True)) + m
        o_ref[...] = out - lse

    return pl.pallas_call(
        body,
        grid=(_N // blk,),
        in_specs=[
            pl.BlockSpec((blk, _D), lambda i: (i, 0)),
            pl.BlockSpec((blk, _D), lambda i: (i, 0)),
            pl.BlockSpec((_D, _D), lambda i: (0, 0)),
            pl.BlockSpec((1, _D), lambda i: (0, 0)),
        ],
        out_specs=pl.BlockSpec((blk, _D), lambda i: (i, 0)),
        out_shape=jax.ShapeDtypeStruct((_N, _D), jnp.float32),
    )(a0, a1, W2, b2.reshape(1, _D))


def kernel(x, adj_or_edge_index, W1, b1, W2, b2):
    row = adj_or_edge_index[0]
    col = adj_or_edge_index[1]
    pad = _EPAD - _E
    row_p = jnp.concatenate([row, jnp.full((pad,), _N, jnp.int32)])
    col_p = jnp.concatenate([col, jnp.zeros((pad,), jnp.int32)])
    row_p = row_p.reshape(_NCHUNK, 1, _K)
    col_p = col_p.reshape(_NCHUNK, _K)
    h = _mlp1(x, W1, b1)
    zeros = jnp.zeros((_RPAD, _D), jnp.float32)
    agg = _sc_aggregate(h, row_p, col_p, zeros)
    return _mlp2(agg[0, :_N], agg[1, :_N], W2, b2)


# speedup vs baseline: 3.2733x; 3.1800x over previous
import functools

import jax
import jax.numpy as jnp
from jax import lax
from jax.experimental import pallas as pl
from jax.experimental.pallas import tpu as pltpu
from jax.experimental.pallas import tpu_sc as plsc

_N, _E, _D = 10000, 320000, 128
_NC, _NS = 2, 16
_NW = _NC * _NS
_K = 128
_NCHUNK = _E // _K
_F_HI = 80
_F_LO = 72
_NHI = 24
_NEPI = _NCHUNK - _NHI * _F_HI - (_NW - _NHI) * _F_LO
_RPAD = 10112
_RPT = _RPAD // _NS


def _mlp1(x, W1, b1):
    blk = 1000

    def body(x_ref, w_ref, b_ref, o_ref):
        h = jnp.dot(x_ref[...], w_ref[...], preferred_element_type=jnp.float32)
        o_ref[...] = jnp.maximum(h + b_ref[...], 0.0)

    return pl.pallas_call(
        body,
        grid=(_N // blk,),
        in_specs=[
            pl.BlockSpec((blk, _D), lambda i: (i, 0)),
            pl.BlockSpec((_D, _D), lambda i: (0, 0)),
            pl.BlockSpec((1, _D), lambda i: (0, 0)),
        ],
        out_specs=pl.BlockSpec((blk, _D), lambda i: (i, 0)),
        out_shape=jax.ShapeDtypeStruct((_N, _D), jnp.float32),
    )(x, W1, b1.reshape(1, _D))


def _sc_aggregate(h, row3, col2, col3):
    mesh = plsc.VectorSubcoreMesh(core_axis_name="c", subcore_axis_name="s")

    @functools.partial(
        pl.kernel,
        mesh=mesh,
        out_type=jax.ShapeDtypeStruct((_NC, _RPAD, _D), jnp.float32),
        scratch_types=[
            pltpu.VMEM((_F_HI, _K), jnp.int32),
            pltpu.VMEM((_K,), jnp.int32),
            pltpu.VMEM((_K,), jnp.int32),
            pltpu.VMEM((_K,), jnp.int32),
            pltpu.VMEM((_K, _D), jnp.float32),
            pltpu.VMEM((_K, _D), jnp.float32),
            pltpu.VMEM_SHARED((_RPAD, _D), jnp.float32),
            pltpu.SemaphoreType.DMA,
            pltpu.SemaphoreType.DMA,
            pltpu.SemaphoreType.DMA,
            pltpu.SemaphoreType.DMA,
            pltpu.SemaphoreType.DMA,
            pltpu.SemaphoreType.DMA,
        ],
    )
    def agg_kernel(h_hbm, row_hbm, col2_hbm, col3_hbm, out_hbm,
                   colv, rowb0, rowb1, colbe, rows0, rows1, acc_sh,
                   gsem0, gsem1, ssem0, ssem1, rsem0, rsem1):
        cid = lax.axis_index("c")
        sid = lax.axis_index("s")
        wid = sid * _NC + cid
        hi = wid < _NHI
        fh = jnp.where(hi, _F_HI // 2, _F_LO // 2)
        gbase = jnp.where(hi, wid * _F_HI,
                          _NHI * _F_HI + (wid - _NHI) * _F_LO)

        def zrow(r, carry):
            for j in range(_D // 16):
                rows0[r, pl.ds(j * 16, 16)] = jnp.zeros((16,), jnp.float32)
            return carry

        lax.fori_loop(0, _K, zrow, 0)
        for k in range(_RPT // _K):
            pltpu.sync_copy(rows0,
                            acc_sh.at[pl.ds(sid * _RPT + k * _K, _K)])
        _tail = _RPT % _K
        pltpu.sync_copy(rows0.at[pl.ds(0, _tail)],
                        acc_sh.at[pl.ds(sid * _RPT + _RPT - _tail, _tail)])

        @pl.when(hi)
        def _():
            pltpu.sync_copy(col2_hbm.at[pl.ds(gbase, _F_HI)], colv)

        @pl.when(jnp.logical_not(hi))
        def _():
            pltpu.sync_copy(col2_hbm.at[pl.ds(gbase, _F_LO)],
                            colv.at[pl.ds(0, _F_LO)])

        plsc.subcore_barrier()

        def rowload(c, rowb, sem):
            return pltpu.async_copy(row_hbm.at[gbase + c, 0], rowb, sem)

        def rowload_wait(c, rowb, sem):
            pltpu.make_async_copy(row_hbm.at[gbase + c, 0], rowb, sem).wait()

        def gather(c, rows, sem):
            pltpu.async_copy(h_hbm.at[colv.at[c, pl.ds(0, 64)]],
                             rows.at[pl.ds(0, 64)], sem)
            pltpu.async_copy(h_hbm.at[colv.at[c, pl.ds(64, 64)]],
                             rows.at[pl.ds(64, 64)], sem)

        def gather_wait(c, rows, sem):
            pltpu.make_async_copy(h_hbm.at[colv.at[c, pl.ds(0, 64)]],
                                  rows.at[pl.ds(0, 64)], sem).wait()
            pltpu.make_async_copy(h_hbm.at[colv.at[c, pl.ds(64, 64)]],
                                  rows.at[pl.ds(64, 64)], sem).wait()

        def scatter(rowb, rows, sem):
            return pltpu.async_copy(rows, acc_sh.at[rowb], sem, add=True)

        def scatter_wait(rowb, rows, sem):
            pltpu.make_async_copy(rows, acc_sh.at[rowb], sem).wait()

        rowload(0, rowb0, rsem0)
        rowload(1, rowb1, rsem1)
        gather(0, rows0, gsem0)

        def body(i, carry):
            c0 = 2 * i
            c1 = c0 + 1

            @pl.when(i < fh)
            def _():
                gather_wait(c0, rows0, gsem0)
                gather(c1, rows1, gsem1)
                rowload_wait(c0, rowb0, rsem0)
                scatter(rowb0, rows0, ssem0)
                gather_wait(c1, rows1, gsem1)
                scatter_wait(rowb0, rows0, ssem0)

                @pl.when(i < fh - 1)
                def _():
                    rowload(c0 + 2, rowb0, rsem0)
                    gather(c0 + 2, rows0, gsem0)

                rowload_wait(c1, rowb1, rsem1)
                scatter(rowb1, rows1, ssem1)
                scatter_wait(rowb1, rows1, ssem1)

                @pl.when(i < fh - 1)
                def _():
                    rowload(c1 + 2, rowb1, rsem1)

            return carry

        lax.fori_loop(0, _F_HI // 2, body, 0)

        @pl.when(wid < _NEPI)
        def _():
            g = _NHI * _F_HI + (_NW - _NHI) * _F_LO + wid
            pltpu.sync_copy(col3_hbm.at[g, 0], colbe)
            pltpu.sync_copy(row_hbm.at[g, 0], rowb0)
            pltpu.async_copy(h_hbm.at[colbe], rows0, gsem0).wait()
            scatter(rowb0, rows0, ssem0)
            scatter_wait(rowb0, rows0, ssem0)

        plsc.subcore_barrier()
        pltpu.sync_copy(acc_sh.at[pl.ds(sid * _RPT, _RPT)],
                        out_hbm.at[cid, pl.ds(sid * _RPT, _RPT)])

    return agg_kernel(h, row3, col2, col3)


def _mlp2(agg, W2, b2):
    blk = 1000

    def body(a_ref, w_ref, b_ref, o_ref):
        a = a_ref[0] + a_ref[1]
        out = jnp.dot(a, w_ref[...], preferred_element_type=jnp.float32)
        out = out + b_ref[...]
        m = jnp.max(out, axis=1, keepdims=True)
        lse = jnp.log(jnp.sum(jnp.exp(out - m), axis=1, keepdims=True)) + m
        o_ref[...] = out - lse

    return pl.pallas_call(
        body,
        grid=(_N // blk,),
        in_specs=[
            pl.BlockSpec((_NC, blk, _D), lambda i: (0, i, 0)),
            pl.BlockSpec((_D, _D), lambda i: (0, 0)),
            pl.BlockSpec((1, _D), lambda i: (0, 0)),
        ],
        out_specs=pl.BlockSpec((blk, _D), lambda i: (i, 0)),
        out_shape=jax.ShapeDtypeStruct((_N, _D), jnp.float32),
    )(agg, W2, b2.reshape(1, _D))


def kernel(x, adj_or_edge_index, W1, b1, W2, b2):
    row3 = adj_or_edge_index[0].reshape(_NCHUNK, 1, _K)
    col2 = adj_or_edge_index[1].reshape(_NCHUNK, _K)
    col3 = adj_or_edge_index[1].reshape(_NCHUNK, 1, _K)
    h = _mlp1(x, W1, b1)
    agg = _sc_aggregate(h, row3, col2, col3)
    return _mlp2(agg, W2, b2)


# speedup vs baseline: 3.3179x; 1.0136x over previous
import functools

import jax
import jax.numpy as jnp
from jax import lax
from jax.experimental import pallas as pl
from jax.experimental.pallas import tpu as pltpu
from jax.experimental.pallas import tpu_sc as plsc

_N, _E, _D = 10000, 320000, 128
_NC, _NS = 2, 16
_NW = _NC * _NS
_K = 128
_NCHUNK = _E // _K
_F_HI = 80
_F_LO = 72
_NHI = 24
_NEPI = _NCHUNK - _NHI * _F_HI - (_NW - _NHI) * _F_LO
_RPAD = 10112
_RPT = _RPAD // _NS


def _mlp1(x, W1, b1):
    blk = 1000

    def body(x_ref, w_ref, b_ref, o_ref):
        h = jnp.dot(x_ref[...], w_ref[...], preferred_element_type=jnp.float32)
        o_ref[...] = jnp.maximum(h + b_ref[...], 0.0)

    return pl.pallas_call(
        body,
        grid=(_N // blk,),
        in_specs=[
            pl.BlockSpec((blk, _D), lambda i: (i, 0)),
            pl.BlockSpec((_D, _D), lambda i: (0, 0)),
            pl.BlockSpec((1, _D), lambda i: (0, 0)),
        ],
        out_specs=pl.BlockSpec((blk, _D), lambda i: (i, 0)),
        out_shape=jax.ShapeDtypeStruct((_N, _D), jnp.float32),
    )(x, W1, b1.reshape(1, _D))


def _sc_aggregate(h, row1, col1):
    mesh = plsc.VectorSubcoreMesh(core_axis_name="c", subcore_axis_name="s")

    @functools.partial(
        pl.kernel,
        mesh=mesh,
        out_type=jax.ShapeDtypeStruct((_NC, _RPAD, _D), jnp.float32),
        scratch_types=[
            pltpu.VMEM((_F_HI * _K,), jnp.int32),
            pltpu.VMEM((_K,), jnp.int32),
            pltpu.VMEM((_K,), jnp.int32),
            pltpu.VMEM((_K,), jnp.int32),
            pltpu.VMEM((_K, _D), jnp.float32),
            pltpu.VMEM((_K, _D), jnp.float32),
            pltpu.VMEM_SHARED((_RPAD, _D), jnp.float32),
            pltpu.SemaphoreType.DMA,
            pltpu.SemaphoreType.DMA,
            pltpu.SemaphoreType.DMA,
            pltpu.SemaphoreType.DMA,
            pltpu.SemaphoreType.DMA,
            pltpu.SemaphoreType.DMA,
        ],
    )
    def agg_kernel(h_hbm, row_hbm, col_hbm, out_hbm,
                   colv, rowb0, rowb1, colbe, rows0, rows1, acc_sh,
                   gsem0, gsem1, ssem0, ssem1, rsem0, rsem1):
        cid = lax.axis_index("c")
        sid = lax.axis_index("s")
        wid = sid * _NC + cid
        hi = wid < _NHI
        fh = jnp.where(hi, _F_HI // 2, _F_LO // 2)
        gbase = jnp.where(hi, wid * _F_HI,
                          _NHI * _F_HI + (wid - _NHI) * _F_LO)

        def zrow(r, carry):
            for j in range(_D // 16):
                rows0[r, pl.ds(j * 16, 16)] = jnp.zeros((16,), jnp.float32)
            return carry

        lax.fori_loop(0, _K, zrow, 0)
        for k in range(_RPT // _K):
            pltpu.sync_copy(rows0,
                            acc_sh.at[pl.ds(sid * _RPT + k * _K, _K)])
        _tail = _RPT % _K
        pltpu.sync_copy(rows0.at[pl.ds(0, _tail)],
                        acc_sh.at[pl.ds(sid * _RPT + _RPT - _tail, _tail)])

        @pl.when(hi)
        def _():
            pltpu.sync_copy(col_hbm.at[pl.ds(gbase * _K, _F_HI * _K)], colv)

        @pl.when(jnp.logical_not(hi))
        def _():
            pltpu.sync_copy(col_hbm.at[pl.ds(gbase * _K, _F_LO * _K)],
                            colv.at[pl.ds(0, _F_LO * _K)])

        plsc.subcore_barrier()

        def rowload(c, rowb, sem):
            return pltpu.async_copy(
                row_hbm.at[pl.ds((gbase + c) * _K, _K)], rowb, sem)

        def rowload_wait(c, rowb, sem):
            pltpu.make_async_copy(
                row_hbm.at[pl.ds((gbase + c) * _K, _K)], rowb, sem).wait()

        def gather(c, rows, sem):
            pltpu.async_copy(h_hbm.at[colv.at[pl.ds(c * _K, 64)]],
                             rows.at[pl.ds(0, 64)], sem)
            pltpu.async_copy(h_hbm.at[colv.at[pl.ds(c * _K + 64, 64)]],
                             rows.at[pl.ds(64, 64)], sem)

        def gather_wait(c, rows, sem):
            pltpu.make_async_copy(h_hbm.at[colv.at[pl.ds(c * _K, 64)]],
                                  rows.at[pl.ds(0, 64)], sem).wait()
            pltpu.make_async_copy(h_hbm.at[colv.at[pl.ds(c * _K + 64, 64)]],
                                  rows.at[pl.ds(64, 64)], sem).wait()

        def scatter(rowb, rows, sem):
            return pltpu.async_copy(rows, acc_sh.at[rowb], sem, add=True)

        def scatter_wait(rowb, rows, sem):
            pltpu.make_async_copy(rows, acc_sh.at[rowb], sem).wait()

        rowload(0, rowb0, rsem0)
        rowload(1, rowb1, rsem1)
        gather(0, rows0, gsem0)

        def body(i, carry):
            c0 = 2 * i
            c1 = c0 + 1

            @pl.when(i < fh)
            def _():
                gather_wait(c0, rows0, gsem0)

                @pl.when(i > 0)
                def _():
                    scatter_wait(rowb1, rows1, ssem1)

                gather(c1, rows1, gsem1)
                rowload_wait(c0, rowb0, rsem0)
                scatter(rowb0, rows0, ssem0)
                gather_wait(c1, rows1, gsem1)
                scatter_wait(rowb0, rows0, ssem0)

                @pl.when(i < fh - 1)
                def _():
                    rowload(c0 + 2, rowb0, rsem0)
                    gather(c0 + 2, rows0, gsem0)

                rowload_wait(c1, rowb1, rsem1)
                scatter(rowb1, rows1, ssem1)

                @pl.when(i < fh - 1)
                def _():
                    rowload(c1 + 2, rowb1, rsem1)

            return carry

        lax.fori_loop(0, _F_HI // 2, body, 0)
        scatter_wait(rowb1, rows1, ssem1)

        @pl.when(wid < _NEPI)
        def _():
            g = _NHI * _F_HI + (_NW - _NHI) * _F_LO + wid
            pltpu.sync_copy(col_hbm.at[pl.ds(g * _K, _K)], colbe)
            pltpu.sync_copy(row_hbm.at[pl.ds(g * _K, _K)], rowb0)
            pltpu.async_copy(h_hbm.at[colbe], rows0, gsem0).wait()
            scatter(rowb0, rows0, ssem0)
            scatter_wait(rowb0, rows0, ssem0)

        plsc.subcore_barrier()
        pltpu.sync_copy(acc_sh.at[pl.ds(sid * _RPT, _RPT)],
                        out_hbm.at[cid, pl.ds(sid * _RPT, _RPT)])

    return agg_kernel(h, row1, col1)


def _mlp2(agg, W2, b2):
    blk = 1000

    def body(a_ref, w_ref, b_ref, o_ref):
        a = a_ref[0] + a_ref[1]
        out = jnp.dot(a, w_ref[...], preferred_element_type=jnp.float32)
        out = out + b_ref[...]
        m = jnp.max(out, axis=1, keepdims=True)
        lse = jnp.log(jnp.sum(jnp.exp(out - m), axis=1, keepdims=True)) + m
        o_ref[...] = out - lse

    return pl.pallas_call(
        body,
        grid=(_N // blk,),
        in_specs=[
            pl.BlockSpec((_NC, blk, _D), lambda i: (0, i, 0)),
            pl.BlockSpec((_D, _D), lambda i: (0, 0)),
            pl.BlockSpec((1, _D), lambda i: (0, 0)),
        ],
        out_specs=pl.BlockSpec((blk, _D), lambda i: (i, 0)),
        out_shape=jax.ShapeDtypeStruct((_N, _D), jnp.float32),
    )(agg, W2, b2.reshape(1, _D))


def kernel(x, adj_or_edge_index, W1, b1, W2, b2):
    row1 = adj_or_edge_index[0]
    col1 = adj_or_edge_index[1]
    h = _mlp1(x, W1, b1)
    agg = _sc_aggregate(h, row1, col1)
    return _mlp2(agg, W2, b2)


# speedup vs baseline: 3.4040x; 1.0260x over previous
import functools

import jax
import jax.numpy as jnp
from jax import lax
from jax.experimental import pallas as pl
from jax.experimental.pallas import tpu as pltpu
from jax.experimental.pallas import tpu_sc as plsc

_N, _E, _D = 10000, 320000, 128
_NC, _NS = 2, 16
_NW = _NC * _NS
_K = 128
_NCHUNK = _E // _K
_F_HI = 80
_F_LO = 72
_NHI = 24
_NEPI = _NCHUNK - _NHI * _F_HI - (_NW - _NHI) * _F_LO
_RPAD = 10112
_RPT = _RPAD // _NS


def _mlp1(x, W1, b1):
    blk = 2000

    def body(x_ref, w_ref, b_ref, o_ref):
        h = jnp.dot(x_ref[...], w_ref[...], preferred_element_type=jnp.float32)
        o_ref[...] = jnp.maximum(h + b_ref[...], 0.0)

    return pl.pallas_call(
        body,
        grid=(_N // blk,),
        in_specs=[
            pl.BlockSpec((blk, _D), lambda i: (i, 0)),
            pl.BlockSpec((_D, _D), lambda i: (0, 0)),
            pl.BlockSpec((1, _D), lambda i: (0, 0)),
        ],
        out_specs=pl.BlockSpec((blk, _D), lambda i: (i, 0)),
        out_shape=jax.ShapeDtypeStruct((_N, _D), jnp.float32),
    )(x, W1, b1.reshape(1, _D))


def _sc_aggregate(h, row1, col1):
    mesh = plsc.VectorSubcoreMesh(core_axis_name="c", subcore_axis_name="s")

    @functools.partial(
        pl.kernel,
        mesh=mesh,
        out_type=jax.ShapeDtypeStruct((_NC, _RPAD, _D), jnp.float32),
        scratch_types=[
            pltpu.VMEM((_F_HI * _K,), jnp.int32),
            pltpu.VMEM((_K,), jnp.int32),
            pltpu.VMEM((_K,), jnp.int32),
            pltpu.VMEM((_K,), jnp.int32),
            pltpu.VMEM((_K, _D), jnp.float32),
            pltpu.VMEM((_K, _D), jnp.float32),
            pltpu.VMEM_SHARED((_RPAD, _D), jnp.float32),
            pltpu.SemaphoreType.DMA,
            pltpu.SemaphoreType.DMA,
            pltpu.SemaphoreType.DMA,
            pltpu.SemaphoreType.DMA,
            pltpu.SemaphoreType.DMA,
            pltpu.SemaphoreType.DMA,
        ],
    )
    def agg_kernel(h_hbm, row_hbm, col_hbm, out_hbm,
                   colv, rowb0, rowb1, colbe, rows0, rows1, acc_sh,
                   gsem0, gsem1, ssem0, ssem1, rsem0, rsem1):
        cid = lax.axis_index("c")
        sid = lax.axis_index("s")
        wid = sid * _NC + cid
        hi = wid < _NHI
        fh = jnp.where(hi, _F_HI // 2, _F_LO // 2)
        gbase = jnp.where(hi, wid * _F_HI,
                          _NHI * _F_HI + (wid - _NHI) * _F_LO)

        def zrow(r, carry):
            for j in range(_D // 16):
                rows0[r, pl.ds(j * 16, 16)] = jnp.zeros((16,), jnp.float32)
            return carry

        lax.fori_loop(0, _K, zrow, 0)
        for k in range(_RPT // _K):
            pltpu.sync_copy(rows0,
                            acc_sh.at[pl.ds(sid * _RPT + k * _K, _K)])
        _tail = _RPT % _K
        pltpu.sync_copy(rows0.at[pl.ds(0, _tail)],
                        acc_sh.at[pl.ds(sid * _RPT + _RPT - _tail, _tail)])

        @pl.when(hi)
        def _():
            pltpu.sync_copy(col_hbm.at[pl.ds(gbase * _K, _F_HI * _K)], colv)

        @pl.when(jnp.logical_not(hi))
        def _():
            pltpu.sync_copy(col_hbm.at[pl.ds(gbase * _K, _F_LO * _K)],
                            colv.at[pl.ds(0, _F_LO * _K)])

        plsc.subcore_barrier()

        def rowload(c, rowb, sem):
            return pltpu.async_copy(
                row_hbm.at[pl.ds((gbase + c) * _K, _K)], rowb, sem)

        def rowload_wait(c, rowb, sem):
            pltpu.make_async_copy(
                row_hbm.at[pl.ds((gbase + c) * _K, _K)], rowb, sem).wait()

        def gather(c, rows, sem):
            pltpu.async_copy(h_hbm.at[colv.at[pl.ds(c * _K, 64)]],
                             rows.at[pl.ds(0, 64)], sem)
            pltpu.async_copy(h_hbm.at[colv.at[pl.ds(c * _K + 64, 64)]],
                             rows.at[pl.ds(64, 64)], sem)

        def gather_wait(c, rows, sem):
            pltpu.make_async_copy(h_hbm.at[colv.at[pl.ds(c * _K, 64)]],
                                  rows.at[pl.ds(0, 64)], sem).wait()
            pltpu.make_async_copy(h_hbm.at[colv.at[pl.ds(c * _K + 64, 64)]],
                                  rows.at[pl.ds(64, 64)], sem).wait()

        def scatter(rowb, rows, sem):
            return pltpu.async_copy(rows, acc_sh.at[rowb], sem, add=True)

        def scatter_wait(rowb, rows, sem):
            pltpu.make_async_copy(rows, acc_sh.at[rowb], sem).wait()

        rowload(0, rowb0, rsem0)
        rowload(1, rowb1, rsem1)
        gather(0, rows0, gsem0)

        def body(i, carry):
            c0 = 2 * i
            c1 = c0 + 1

            @pl.when(i < fh)
            def _():
                gather_wait(c0, rows0, gsem0)

                @pl.when(i > 0)
                def _():
                    scatter_wait(rowb1, rows1, ssem1)

                gather(c1, rows1, gsem1)
                rowload_wait(c0, rowb0, rsem0)
                scatter(rowb0, rows0, ssem0)
                gather_wait(c1, rows1, gsem1)
                scatter_wait(rowb0, rows0, ssem0)

                @pl.when(i < fh - 1)
                def _():
                    rowload(c0 + 2, rowb0, rsem0)
                    gather(c0 + 2, rows0, gsem0)

                rowload_wait(c1, rowb1, rsem1)
                scatter(rowb1, rows1, ssem1)

                @pl.when(i < fh - 1)
                def _():
                    rowload(c1 + 2, rowb1, rsem1)

            return carry

        lax.fori_loop(0, _F_HI // 2, body, 0)
        scatter_wait(rowb1, rows1, ssem1)

        @pl.when(wid < _NEPI)
        def _():
            g = _NHI * _F_HI + (_NW - _NHI) * _F_LO + wid
            pltpu.sync_copy(col_hbm.at[pl.ds(g * _K, _K)], colbe)
            pltpu.sync_copy(row_hbm.at[pl.ds(g * _K, _K)], rowb0)
            pltpu.async_copy(h_hbm.at[colbe], rows0, gsem0).wait()
            scatter(rowb0, rows0, ssem0)
            scatter_wait(rowb0, rows0, ssem0)

        plsc.subcore_barrier()
        pltpu.sync_copy(acc_sh.at[pl.ds(sid * _RPT, _RPT)],
                        out_hbm.at[cid, pl.ds(sid * _RPT, _RPT)])

    return agg_kernel(h, row1, col1)


def _mlp2(agg, W2, b2):
    blk = 2000

    def body(a_ref, w_ref, b_ref, o_ref):
        a = a_ref[0] + a_ref[1]
        out = jnp.dot(a, w_ref[...], preferred_element_type=jnp.float32)
        out = out + b_ref[...]
        m = jnp.max(out, axis=1, keepdims=True)
        lse = jnp.log(jnp.sum(jnp.exp(out - m), axis=1, keepdims=True)) + m
        o_ref[...] = out - lse

    return pl.pallas_call(
        body,
        grid=(_N // blk,),
        in_specs=[
            pl.BlockSpec((_NC, blk, _D), lambda i: (0, i, 0)),
            pl.BlockSpec((_D, _D), lambda i: (0, 0)),
            pl.BlockSpec((1, _D), lambda i: (0, 0)),
        ],
        out_specs=pl.BlockSpec((blk, _D), lambda i: (i, 0)),
        out_shape=jax.ShapeDtypeStruct((_N, _D), jnp.float32),
    )(agg, W2, b2.reshape(1, _D))


def kernel(x, adj_or_edge_index, W1, b1, W2, b2):
    row1 = adj_or_edge_index[0]
    col1 = adj_or_edge_index[1]
    h = _mlp1(x, W1, b1)
    agg = _sc_aggregate(h, row1, col1)
    return _mlp2(agg, W2, b2)
